# baseline (speedup 1.0000x reference)
import functools

import jax
import jax.numpy as jnp
from jax import lax
from jax.experimental import pallas as pl
from jax.experimental.pallas import tpu as pltpu
from jax.experimental.pallas import tpu_sc as plsc

BINS = 256
HEIGHT_RATE = 0.5

B = 8
H = 224
W = 224
PLANES = 96
NC = 2
NS = 16
NW = NC * NS
PPW = (B * PLANES) // NW
LANES = 16
HSTRIDE = BINS + 1
VEC_PER_ROW = W // LANES
PLANE_BLK = 48


def _sc_hist_body(x_hbm, out_hbm, buf0, buf1, hist, acc, sem0, sem1):
    wid = lax.axis_index("s") * NC + lax.axis_index("c")

    zeros16 = jnp.zeros((LANES,), jnp.float32)
    ones16 = jnp.ones((LANES,), jnp.float32)
    lane_iota = lax.iota(jnp.int32, LANES)
    lane_base = lane_iota * HSTRIDE

    def zbody(i, carry):
        hist[pl.ds(i * LANES, LANES)] = zeros16
        return carry

    lax.fori_loop(0, HSTRIDE, zbody, 0)

    def process(buf):
        def body(g, carry):
            bis = []
            for k in range(2):
                for c in range(VEC_PER_ROW):
                    v = buf[g * 2 + k, pl.ds(c * LANES, LANES)]
                    bis.append((v * jnp.float32(BINS)).astype(jnp.int32) + lane_base)
            for bi in bis:
                plsc.addupdate_scatter(hist, [bi], ones16)
            return carry

        lax.fori_loop(0, H // 2, body, 0, unroll=4)

    pltpu.async_copy(x_hbm.at[wid, 0], buf0, sem0)

    def pair_body(p, carry):
        c0 = p * 2
        pltpu.async_copy(x_hbm.at[wid, c0 + 1], buf1, sem1)
        pltpu.make_async_copy(x_hbm.at[wid, 0], buf0, sem0).wait()
        process(buf0)

        @pl.when(c0 + 2 < PPW)
        def _():
            pltpu.async_copy(x_hbm.at[wid, c0 + 2], buf0, sem0)

        pltpu.make_async_copy(x_hbm.at[wid, 0], buf1, sem1).wait()
        process(buf1)
        return carry

    lax.fori_loop(0, PPW // 2, pair_body, 0)

    for g in range(BINS // LANES):
        s = zeros16
        for r in range(LANES):
            s = s + plsc.load_gather(hist, [lane_iota + (r * HSTRIDE + g * LANES)])
        acc[pl.ds(g * LANES, LANES)] = s

    pltpu.sync_copy(acc, out_hbm.at[wid % 4, wid // 4])


def _sc_hist(x4):
    mesh = plsc.VectorSubcoreMesh(core_axis_name="c", subcore_axis_name="s")
    k = functools.partial(
        pl.kernel,
        mesh=mesh,
        compiler_params=pltpu.CompilerParams(needs_layout_passes=False),
        out_type=jax.ShapeDtypeStruct((4, B, BINS), jnp.float32),
        scratch_types=[
            pltpu.VMEM((H, W), jnp.float32),
            pltpu.VMEM((H, W), jnp.float32),
            pltpu.VMEM((LANES * HSTRIDE,), jnp.float32),
            pltpu.VMEM((BINS,), jnp.float32),
            pltpu.SemaphoreType.DMA,
            pltpu.SemaphoreType.DMA,
        ],
    )(_sc_hist_body)
    return k(x4)


def _tc_head_body(p_ref, w1, b1, w2, b2, w3, b3, w4t, b4b, w_out, val_out):
    hst = p_ref[0] + p_ref[1] + p_ref[2] + p_ref[3]
    m = jnp.max(hst, axis=1, keepdims=True)
    ar = lax.broadcasted_iota(jnp.int32, (B, BINS), 1)
    big = jnp.int32(BINS)
    bin_idx = jnp.min(jnp.where(hst == m, ar, big), axis=1, keepdims=True)
    cond = jnp.logical_or(ar < bin_idx, hst > m * HEIGHT_RATE)
    zmin = jnp.min(jnp.where(cond, big, ar), axis=1)
    value_idx = jnp.where(zmin == big, 0, zmin)
    val = value_idx.astype(jnp.float32) * (1.0 / BINS)
    val_out[...] = jnp.broadcast_to(val[:, None], (B, 128))

    h = jnp.maximum(jnp.dot(hst, w1[...], preferred_element_type=jnp.float32) + b1[...], 0.0)
    h = jnp.maximum(jnp.dot(h, w2[...], preferred_element_type=jnp.float32) + b2[...], 0.0)
    h = jnp.maximum(jnp.dot(h, w3[...], preferred_element_type=jnp.float32) + b3[...], 0.0)
    w = jnp.sum(h * w4t[...], axis=1, keepdims=True)
    w_out[...] = jnp.broadcast_to(w, (B, 128)) + b4b[...]


def _tc_head(partials, W1, b1r, W2, b2r, W3, b3r, W4T, b4b):
    return pl.pallas_call(
        _tc_head_body,
        out_shape=[
            jax.ShapeDtypeStruct((B, 128), jnp.float32),
            jax.ShapeDtypeStruct((B, 128), jnp.float32),
        ],
    )(partials, W1, b1r, W2, b2r, W3, b3r, W4T, b4b)


def _rescale_body(w_ref, x_ref, o_ref):
    b = pl.program_id(0)
    o_ref[...] = x_ref[...] * w_ref[b]


def _rescale(x, w_vec):
    grid = (B, PLANES // PLANE_BLK)
    return pl.pallas_call(
        _rescale_body,
        grid=grid,
        in_specs=[
            pl.BlockSpec(memory_space=pltpu.SMEM),
            pl.BlockSpec((1, PLANE_BLK, H, W), lambda b, i: (b, i, 0, 0)),
        ],
        out_specs=pl.BlockSpec((1, PLANE_BLK, H, W), lambda b, i: (b, i, 0, 0)),
        out_shape=jax.ShapeDtypeStruct((B, PLANES, H, W), jnp.float32),
    )(w_vec, x)


def kernel(x, W1, b1, W2, b2, W3, b3, W4, b4):
    x4 = x.reshape(NW, PPW, H, W)
    partials = _sc_hist(x4)

    b1r = b1.reshape(1, 32)
    b2r = b2.reshape(1, 64)
    b3r = b3.reshape(1, 128)
    W4T = W4.reshape(1, 128)
    b4b = jnp.broadcast_to(b4.reshape(1, 1), (1, 128))
    w_full, val_full = _tc_head(partials, W1, b1r, W2, b2r, W3, b3r, W4T, b4b)
    w_vec = w_full[:, 0]
    value = val_full[:, 0]

    out = _rescale(x, w_vec)
    return (out, value)


# speedup vs baseline: 1.0040x; 1.0040x over previous
import functools

import jax
import jax.numpy as jnp
from jax import lax
from jax.experimental import pallas as pl
from jax.experimental.pallas import tpu as pltpu
from jax.experimental.pallas import tpu_sc as plsc

BINS = 256
HEIGHT_RATE = 0.5

B = 8
H = 224
W = 224
PLANES = 96
NC = 2
NS = 16
NW = NC * NS
PPW = (B * PLANES) // NW
LANES = 16
HSTRIDE = BINS + 1
VEC_PER_ROW = W // LANES
PLANE_BLK = 24


def _sc_hist_body(x_hbm, out_hbm, buf0, buf1, hist, acc, sem0, sem1):
    wid = lax.axis_index("s") * NC + lax.axis_index("c")

    zeros16 = jnp.zeros((LANES,), jnp.float32)
    ones16 = jnp.ones((LANES,), jnp.float32)
    lane_iota = lax.iota(jnp.int32, LANES)
    lane_base = lane_iota * HSTRIDE

    def zbody(i, carry):
        hist[pl.ds(i * LANES, LANES)] = zeros16
        return carry

    lax.fori_loop(0, HSTRIDE, zbody, 0)

    def process(buf):
        def body(g, carry):
            bis = []
            for k in range(2):
                for c in range(VEC_PER_ROW):
                    v = buf[g * 2 + k, pl.ds(c * LANES, LANES)]
                    bis.append((v * jnp.float32(BINS)).astype(jnp.int32) + lane_base)
            for bi in bis:
                plsc.addupdate_scatter(hist, [bi], ones16)
            return carry

        lax.fori_loop(0, H // 2, body, 0, unroll=2)

    pltpu.async_copy(x_hbm.at[wid, 0], buf0, sem0)

    def pair_body(p, carry):
        c0 = p * 2
        pltpu.async_copy(x_hbm.at[wid, c0 + 1], buf1, sem1)
        pltpu.make_async_copy(x_hbm.at[wid, 0], buf0, sem0).wait()
        process(buf0)

        @pl.when(c0 + 2 < PPW)
        def _():
            pltpu.async_copy(x_hbm.at[wid, c0 + 2], buf0, sem0)

        pltpu.make_async_copy(x_hbm.at[wid, 0], buf1, sem1).wait()
        process(buf1)
        return carry

    lax.fori_loop(0, PPW // 2, pair_body, 0)

    for g in range(BINS // LANES):
        s = zeros16
        for r in range(LANES):
            s = s + plsc.load_gather(hist, [lane_iota + (r * HSTRIDE + g * LANES)])
        acc[pl.ds(g * LANES, LANES)] = s

    pltpu.sync_copy(acc, out_hbm.at[wid % 4, wid // 4])


def _sc_hist(x4):
    mesh = plsc.VectorSubcoreMesh(core_axis_name="c", subcore_axis_name="s")
    k = functools.partial(
        pl.kernel,
        mesh=mesh,
        compiler_params=pltpu.CompilerParams(needs_layout_passes=False),
        out_type=jax.ShapeDtypeStruct((4, B, BINS), jnp.float32),
        scratch_types=[
            pltpu.VMEM((H, W), jnp.float32),
            pltpu.VMEM((H, W), jnp.float32),
            pltpu.VMEM((LANES * HSTRIDE,), jnp.float32),
            pltpu.VMEM((BINS,), jnp.float32),
            pltpu.SemaphoreType.DMA,
            pltpu.SemaphoreType.DMA,
        ],
    )(_sc_hist_body)
    return k(x4)


def _tc_head_body(p_ref, w1, b1, w2, b2, w3, b3, w4t, b4b, w_out, val_out):
    hst = p_ref[0] + p_ref[1] + p_ref[2] + p_ref[3]
    m = jnp.max(hst, axis=1, keepdims=True)
    ar = lax.broadcasted_iota(jnp.int32, (B, BINS), 1)
    big = jnp.int32(BINS)
    bin_idx = jnp.min(jnp.where(hst == m, ar, big), axis=1, keepdims=True)
    cond = jnp.logical_or(ar < bin_idx, hst > m * HEIGHT_RATE)
    zmin = jnp.min(jnp.where(cond, big, ar), axis=1)
    value_idx = jnp.where(zmin == big, 0, zmin)
    val = value_idx.astype(jnp.float32) * (1.0 / BINS)
    val_out[...] = jnp.broadcast_to(val[:, None], (B, 128))

    h = jnp.maximum(jnp.dot(hst, w1[...], preferred_element_type=jnp.float32) + b1[...], 0.0)
    h = jnp.maximum(jnp.dot(h, w2[...], preferred_element_type=jnp.float32) + b2[...], 0.0)
    h = jnp.maximum(jnp.dot(h, w3[...], preferred_element_type=jnp.float32) + b3[...], 0.0)
    w = jnp.sum(h * w4t[...], axis=1, keepdims=True)
    w_out[...] = jnp.broadcast_to(w, (B, 128)) + b4b[...]


def _tc_head(partials, W1, b1r, W2, b2r, W3, b3r, W4T, b4b):
    return pl.pallas_call(
        _tc_head_body,
        out_shape=[
            jax.ShapeDtypeStruct((B, 128), jnp.float32),
            jax.ShapeDtypeStruct((B, 128), jnp.float32),
        ],
    )(partials, W1, b1r, W2, b2r, W3, b3r, W4T, b4b)


def _rescale_body(w_ref, x_ref, o_ref):
    b = pl.program_id(0)
    o_ref[...] = x_ref[...] * w_ref[b]


def _rescale(x, w_vec):
    grid = (B, PLANES // PLANE_BLK)
    return pl.pallas_call(
        _rescale_body,
        grid=grid,
        in_specs=[
            pl.BlockSpec(memory_space=pltpu.SMEM),
            pl.BlockSpec((1, PLANE_BLK, H, W), lambda b, i: (b, i, 0, 0)),
        ],
        out_specs=pl.BlockSpec((1, PLANE_BLK, H, W), lambda b, i: (b, i, 0, 0)),
        out_shape=jax.ShapeDtypeStruct((B, PLANES, H, W), jnp.float32),
    )(w_vec, x)


def kernel(x, W1, b1, W2, b2, W3, b3, W4, b4):
    x4 = x.reshape(NW, PPW, H, W)
    partials = _sc_hist(x4)

    b1r = b1.reshape(1, 32)
    b2r = b2.reshape(1, 64)
    b3r = b3.reshape(1, 128)
    W4T = W4.reshape(1, 128)
    b4b = jnp.broadcast_to(b4.reshape(1, 1), (1, 128))
    w_full, val_full = _tc_head(partials, W1, b1r, W2, b2r, W3, b3r, W4T, b4b)
    w_vec = w_full[:, 0]
    value = val_full[:, 0]

    out = _rescale(x, w_vec)
    return (out, value)
